# baseline (speedup 1.0000x reference)
import functools

import jax
import jax.numpy as jnp
from jax import lax
from jax.experimental import pallas as pl
from jax.experimental.pallas import tpu as pltpu
from jax.experimental.pallas import tpu_sc as plsc

B = 1024
D = 64
V = 100000
MAX_NORM = 1.0

_info = plsc.get_sparse_core_info()
_NC, _NS = _info.num_cores, _info.num_subcores
_NW = _NC * _NS
_BPW = B // _NW


def _sc_gather_body(table_hbm, idx_hbm, out_hbm, idx_v, rows_v, sem):
    wid = lax.axis_index("s") * _NC + lax.axis_index("c")
    base = wid * _BPW
    pltpu.sync_copy(idx_hbm.at[pl.ds(base, _BPW)], idx_v)
    pltpu.async_copy(table_hbm.at[idx_v], rows_v, sem).wait()
    pltpu.sync_copy(rows_v, out_hbm.at[pl.ds(base, _BPW)])


def _gather_rows(emb_table, inputs_):
    mesh = plsc.VectorSubcoreMesh(core_axis_name="c", subcore_axis_name="s")
    fn = pl.kernel(
        _sc_gather_body,
        mesh=mesh,
        out_type=jax.ShapeDtypeStruct((B, D), jnp.float32),
        scratch_types=[
            pltpu.VMEM((_BPW,), jnp.int32),
            pltpu.VMEM((_BPW, D), jnp.float32),
            pltpu.SemaphoreType.DMA,
        ],
        compiler_params=pltpu.CompilerParams(use_tc_tiling_on_sc=False),
    )
    return fn(emb_table, inputs_)


BN = 2048
_NB = pl.cdiv(V, BN)


def _mm_body(e_ref, w_ref, b_ref, out_ref, es_ref):
    @pl.when(pl.program_id(0) == 0)
    def _():
        e = e_ref[...]
        norm = jnp.sqrt(jnp.sum(e * e, axis=1, keepdims=True))
        scale = jnp.minimum(1.0, MAX_NORM / jnp.maximum(norm, 1e-12))
        es_ref[...] = e * scale

    acc = jnp.dot(es_ref[...], w_ref[...], preferred_element_type=jnp.float32)
    out_ref[...] = acc + b_ref[...]


def _project(e, W, b2d):
    return pl.pallas_call(
        _mm_body,
        grid=(_NB,),
        in_specs=[
            pl.BlockSpec((B, D), lambda j: (0, 0)),
            pl.BlockSpec((D, BN), lambda j: (0, j)),
            pl.BlockSpec((1, BN), lambda j: (0, j)),
        ],
        out_specs=pl.BlockSpec((B, BN), lambda j: (0, j)),
        out_shape=jax.ShapeDtypeStruct((B, V), jnp.float32),
        scratch_shapes=[pltpu.VMEM((B, D), jnp.float32)],
    )(e, W, b2d)


@jax.jit
def kernel(inputs_, emb_table, W, b):
    e = _gather_rows(emb_table, inputs_.astype(jnp.int32))
    return _project(e, W, b.reshape(1, V))


# speedup vs baseline: 1.0005x; 1.0005x over previous
import jax
import jax.numpy as jnp
from jax import lax
from jax.experimental import pallas as pl
from jax.experimental.pallas import tpu as pltpu
from jax.experimental.pallas import tpu_sc as plsc

B = 1024
D = 64
V = 100000
MAX_NORM = 1.0

_info = plsc.get_sparse_core_info()
_NC, _NS = _info.num_cores, _info.num_subcores
_NW = _NC * _NS
_BPW = B // _NW


def _sc_gather_body(table_hbm, idx_hbm, out_hbm, idx_v, idxh_v, rows_v, sem):
    wid = lax.axis_index("s") * _NC + lax.axis_index("c")
    base = wid * _BPW
    pltpu.sync_copy(idx_hbm.at[pl.ds(base, _BPW)], idx_v)
    for c in range(_BPW // 16):
        sl = pl.ds(16 * c, 16)
        idxh_v[sl] = lax.shift_right_logical(idx_v[sl], 1)
    pltpu.async_copy(table_hbm.at[idxh_v], rows_v, sem).wait()
    pltpu.sync_copy(rows_v, out_hbm.at[pl.ds(base, _BPW)])


def _gather_rows(table2, inputs_):
    mesh = plsc.VectorSubcoreMesh(core_axis_name="c", subcore_axis_name="s")
    fn = pl.kernel(
        _sc_gather_body,
        mesh=mesh,
        out_type=jax.ShapeDtypeStruct((B, 2 * D), jnp.float32),
        scratch_types=[
            pltpu.VMEM((_BPW,), jnp.int32),
            pltpu.VMEM((_BPW,), jnp.int32),
            pltpu.VMEM((_BPW, 2 * D), jnp.float32),
            pltpu.SemaphoreType.DMA,
        ],
    )
    return fn(table2, inputs_)


BN = 2048
_NB = pl.cdiv(V, BN)


def _mm_body(e_ref, idx_ref, w_ref, b_ref, out_ref, es_ref):
    @pl.when(pl.program_id(0) == 0)
    def _():
        odd = (idx_ref[...] & 1) == 1
        e = jnp.where(odd, e_ref[:, D:], e_ref[:, :D])
        norm = jnp.sqrt(jnp.sum(e * e, axis=1, keepdims=True))
        scale = jnp.minimum(1.0, MAX_NORM / jnp.maximum(norm, 1e-12))
        es_ref[...] = e * scale

    acc = jnp.dot(es_ref[...], w_ref[...], preferred_element_type=jnp.float32)
    out_ref[...] = acc + b_ref[...]


def _project(e2, idx2d, W, b2d):
    return pl.pallas_call(
        _mm_body,
        grid=(_NB,),
        in_specs=[
            pl.BlockSpec((B, 2 * D), lambda j: (0, 0)),
            pl.BlockSpec((B, 1), lambda j: (0, 0)),
            pl.BlockSpec((D, BN), lambda j: (0, j)),
            pl.BlockSpec((1, BN), lambda j: (0, j)),
        ],
        out_specs=pl.BlockSpec((B, BN), lambda j: (0, j)),
        out_shape=jax.ShapeDtypeStruct((B, V), jnp.float32),
        scratch_shapes=[pltpu.VMEM((B, D), jnp.float32)],
    )(e2, idx2d, W, b2d)


@jax.jit
def kernel(inputs_, emb_table, W, b):
    idx = inputs_.astype(jnp.int32)
    table2 = emb_table.reshape(V // 2, 2 * D)
    e2 = _gather_rows(table2, idx)
    return _project(e2, idx.reshape(B, 1), W, b.reshape(1, V))


# speedup vs baseline: 2.0861x; 2.0850x over previous
import jax
import jax.numpy as jnp
from jax import lax
from jax.experimental import pallas as pl
from jax.experimental.pallas import tpu as pltpu
from jax.experimental.pallas import tpu_sc as plsc

B = 1024
D = 64
V = 100000
MAX_NORM = 1.0

_info = plsc.get_sparse_core_info()
_NC, _NS = _info.num_cores, _info.num_subcores
_NW = _NC * _NS
_BPW = B // _NW


def _sc_gather_body(table_hbm, idx_hbm, out_hbm, idx_v, idxh_v, rows_v, sem):
    wid = lax.axis_index("s") * _NC + lax.axis_index("c")
    base = wid * _BPW
    pltpu.sync_copy(idx_hbm.at[pl.ds(base, _BPW)], idx_v)
    for c in range(_BPW // 16):
        sl = pl.ds(16 * c, 16)
        idxh_v[sl] = lax.shift_right_logical(idx_v[sl], 1)
    pltpu.async_copy(table_hbm.at[idxh_v], rows_v, sem).wait()
    pltpu.sync_copy(rows_v, out_hbm.at[pl.ds(base, _BPW)])


def _gather_rows(table2, inputs_):
    mesh = plsc.VectorSubcoreMesh(core_axis_name="c", subcore_axis_name="s")
    fn = pl.kernel(
        _sc_gather_body,
        mesh=mesh,
        out_type=jax.ShapeDtypeStruct((B, 2 * D), jnp.float32),
        scratch_types=[
            pltpu.VMEM((_BPW,), jnp.int32),
            pltpu.VMEM((_BPW,), jnp.int32),
            pltpu.VMEM((_BPW, 2 * D), jnp.float32),
            pltpu.SemaphoreType.DMA,
        ],
    )
    return fn(table2, inputs_)


BN = 2048
_NB = pl.cdiv(V, BN)


def _mm_body(e_ref, idx_ref, w_ref, b_ref, out_ref, est_ref):
    @pl.when(pl.program_id(0) == 0)
    def _():
        odd = (idx_ref[...] & 1) == 1
        e = jnp.where(odd, e_ref[:, D:], e_ref[:, :D])
        norm = jnp.sqrt(jnp.sum(e * e, axis=1, keepdims=True))
        scale = jnp.minimum(1.0, MAX_NORM / jnp.maximum(norm, 1e-12))
        est_ref[...] = (e * scale).T

    acc = lax.dot_general(
        w_ref[...], est_ref[...],
        dimension_numbers=(((0,), (0,)), ((), ())),
        preferred_element_type=jnp.float32,
    )
    out_ref[...] = acc + b_ref[...]


def _project(e2, idx2d, W, bc):
    return pl.pallas_call(
        _mm_body,
        grid=(_NB,),
        in_specs=[
            pl.BlockSpec((B, 2 * D), lambda j: (0, 0)),
            pl.BlockSpec((B, 1), lambda j: (0, 0)),
            pl.BlockSpec((D, BN), lambda j: (0, j)),
            pl.BlockSpec((BN, 1), lambda j: (j, 0)),
        ],
        out_specs=pl.BlockSpec((BN, B), lambda j: (j, 0)),
        out_shape=jax.ShapeDtypeStruct((V, B), jnp.float32),
        scratch_shapes=[pltpu.VMEM((D, B), jnp.float32)],
    )(e2, idx2d, W, bc)


@jax.jit
def kernel(inputs_, emb_table, W, b):
    idx = inputs_.astype(jnp.int32)
    table2 = emb_table.reshape(V // 2, 2 * D)
    e2 = _gather_rows(table2, idx)
    logits_t = _project(e2, idx.reshape(B, 1), W, b.reshape(V, 1))
    return logits_t.T


# speedup vs baseline: 2.6211x; 1.2565x over previous
import jax
import jax.numpy as jnp
from jax import lax
from jax.experimental import pallas as pl
from jax.experimental.pallas import tpu as pltpu
from jax.experimental.pallas import tpu_sc as plsc

B = 1024
D = 64
V = 100000
MAX_NORM = 1.0

_info = plsc.get_sparse_core_info()
_NC, _NS = _info.num_cores, _info.num_subcores
_NW = _NC * _NS
_BPW = B // _NW


def _sc_gather_body(table_hbm, idx_hbm, out_hbm, idx_v, idxh_v, rows_v, sem):
    wid = lax.axis_index("s") * _NC + lax.axis_index("c")
    base = wid * _BPW
    pltpu.sync_copy(idx_hbm.at[pl.ds(base, _BPW)], idx_v)
    for c in range(_BPW // 16):
        sl = pl.ds(16 * c, 16)
        idxh_v[sl] = lax.shift_right_logical(idx_v[sl], 1)
    pltpu.async_copy(table_hbm.at[idxh_v], rows_v, sem).wait()
    pltpu.sync_copy(rows_v, out_hbm.at[pl.ds(base, _BPW)])


def _gather_rows(table2, inputs_):
    mesh = plsc.VectorSubcoreMesh(core_axis_name="c", subcore_axis_name="s")
    fn = pl.kernel(
        _sc_gather_body,
        mesh=mesh,
        out_type=jax.ShapeDtypeStruct((B, 2 * D), jnp.float32),
        scratch_types=[
            pltpu.VMEM((_BPW,), jnp.int32),
            pltpu.VMEM((_BPW,), jnp.int32),
            pltpu.VMEM((_BPW, 2 * D), jnp.float32),
            pltpu.SemaphoreType.DMA,
        ],
    )
    return fn(table2, inputs_)


BN = 2048
_NB = pl.cdiv(V, BN)


def _mm_body(e_ref, idx_ref, w_ref, b_ref, out_ref, est_ref):
    @pl.when(pl.program_id(0) == 0)
    def _():
        odd = (idx_ref[...] & 1) == 1
        e = jnp.where(odd, e_ref[:, D:], e_ref[:, :D])
        norm = jnp.sqrt(jnp.sum(e * e, axis=1, keepdims=True))
        scale = jnp.minimum(1.0, MAX_NORM / jnp.maximum(norm, 1e-12))
        est_ref[...] = (e * scale).T.astype(jnp.bfloat16)

    acc = lax.dot_general(
        w_ref[...].astype(jnp.bfloat16), est_ref[...],
        dimension_numbers=(((0,), (0,)), ((), ())),
        preferred_element_type=jnp.float32,
    )
    out_ref[...] = acc + b_ref[...].T


def _project(e2, idx2d, W, br):
    return pl.pallas_call(
        _mm_body,
        grid=(_NB,),
        in_specs=[
            pl.BlockSpec((B, 2 * D), lambda j: (0, 0)),
            pl.BlockSpec((B, 1), lambda j: (0, 0)),
            pl.BlockSpec((D, BN), lambda j: (0, j)),
            pl.BlockSpec((1, BN), lambda j: (0, j)),
        ],
        out_specs=pl.BlockSpec((BN, B), lambda j: (j, 0)),
        out_shape=jax.ShapeDtypeStruct((V, B), jnp.float32),
        scratch_shapes=[pltpu.VMEM((D, B), jnp.bfloat16)],
    )(e2, idx2d, W, br)


@jax.jit
def kernel(inputs_, emb_table, W, b):
    idx = inputs_.astype(jnp.int32)
    table2 = emb_table.reshape(V // 2, 2 * D)
    e2 = _gather_rows(table2, idx)
    logits_t = _project(e2, idx.reshape(B, 1), W, b.reshape(1, V))
    return logits_t.T


# speedup vs baseline: 2.6349x; 1.0053x over previous
import jax
import jax.numpy as jnp
from jax import lax
from jax.experimental import pallas as pl
from jax.experimental.pallas import tpu as pltpu
from jax.experimental.pallas import tpu_sc as plsc

B = 1024
D = 64
V = 100000
MAX_NORM = 1.0

_info = plsc.get_sparse_core_info()
_NC, _NS = _info.num_cores, _info.num_subcores
_NW = _NC * _NS
_BPW = B // _NW


def _sc_gather_body(table_hbm, idx_hbm, out_hbm, idx_v, idxh_v, rows_v, sem):
    wid = lax.axis_index("s") * _NC + lax.axis_index("c")
    base = wid * _BPW
    pltpu.sync_copy(idx_hbm.at[pl.ds(base, _BPW)], idx_v)
    for c in range(_BPW // 16):
        sl = pl.ds(16 * c, 16)
        idxh_v[sl] = lax.shift_right_logical(idx_v[sl], 1)
    pltpu.async_copy(table_hbm.at[idxh_v], rows_v, sem).wait()
    pltpu.sync_copy(rows_v, out_hbm.at[pl.ds(base, _BPW)])


def _gather_rows(table2, inputs_):
    mesh = plsc.VectorSubcoreMesh(core_axis_name="c", subcore_axis_name="s")
    fn = pl.kernel(
        _sc_gather_body,
        mesh=mesh,
        out_type=jax.ShapeDtypeStruct((B, 2 * D), jnp.float32),
        scratch_types=[
            pltpu.VMEM((_BPW,), jnp.int32),
            pltpu.VMEM((_BPW,), jnp.int32),
            pltpu.VMEM((_BPW, 2 * D), jnp.float32),
            pltpu.SemaphoreType.DMA,
        ],
        compiler_params=pltpu.CompilerParams(use_tc_tiling_on_sc=True),
    )
    return fn(table2, inputs_)


BN = 2048
_NB = pl.cdiv(V, BN)


def _mm_body(e_ref, idx_ref, w_ref, b_ref, out_ref, est_ref):
    @pl.when(pl.program_id(0) == 0)
    def _():
        odd = (idx_ref[...] & 1) == 1
        e = jnp.where(odd, e_ref[:, D:], e_ref[:, :D])
        norm = jnp.sqrt(jnp.sum(e * e, axis=1, keepdims=True))
        scale = jnp.minimum(1.0, MAX_NORM / jnp.maximum(norm, 1e-12))
        est_ref[...] = (e * scale).T.astype(jnp.bfloat16)

    acc = lax.dot_general(
        w_ref[...].astype(jnp.bfloat16), est_ref[...],
        dimension_numbers=(((0,), (0,)), ((), ())),
        preferred_element_type=jnp.float32,
    )
    out_ref[...] = acc + b_ref[...].T


def _project(e2, idx2d, W, br):
    return pl.pallas_call(
        _mm_body,
        grid=(_NB,),
        in_specs=[
            pl.BlockSpec((B, 2 * D), lambda j: (0, 0)),
            pl.BlockSpec((B, 1), lambda j: (0, 0)),
            pl.BlockSpec((D, BN), lambda j: (0, j)),
            pl.BlockSpec((1, BN), lambda j: (0, j)),
        ],
        out_specs=pl.BlockSpec((BN, B), lambda j: (j, 0)),
        out_shape=jax.ShapeDtypeStruct((V, B), jnp.float32),
        scratch_shapes=[pltpu.VMEM((D, B), jnp.bfloat16)],
    )(e2, idx2d, W, br)


@jax.jit
def kernel(inputs_, emb_table, W, b):
    idx = inputs_.astype(jnp.int32)
    table2 = emb_table.reshape(V // 2, 2 * D)
    e2 = _gather_rows(table2, idx)
    logits_t = _project(e2, idx.reshape(B, 1), W, b.reshape(1, V))
    return logits_t.T
